# baseline (speedup 1.0000x reference)
import jax
import jax.numpy as jnp
from jax import lax
from jax.experimental import pallas as pl
from jax.experimental.pallas import tpu as pltpu

VOCAB = 100000
EMBED_DIM = 128
CONTEXT = 200
HIDDEN = 128
KDIM = CONTEXT * EMBED_DIM

TILE = 3200
NTILE = VOCAB // TILE
TAIL = VOCAB - NTILE * TILE
NBUF = 10


def _nt_dot(a, b):
    return lax.dot_general(a, b, (((1,), (1,)), ((), ())),
                           preferred_element_type=jnp.float32)


def _mlp_kernel(idx_ref, b1_ref, b2_ref, emb_hbm, w1_hbm, w2_hbm, out_ref,
                emb_v, w1_v, bufs, tail_v, sems, gsem):
    gcp = [None] * CONTEXT
    for c in range(CONTEXT):
        gcp[c] = pltpu.make_async_copy(
            emb_hbm.at[pl.ds(idx_ref[c], 1)],
            emb_v.at[:, pl.ds(c * EMBED_DIM, EMBED_DIM)], gsem)
        gcp[c].start()

    w1_cp = pltpu.make_async_copy(w1_hbm, w1_v, sems.at[NBUF])
    w1_cp.start()
    copies = [None] * NTILE
    for t in range(NBUF):
        copies[t] = pltpu.make_async_copy(
            w2_hbm.at[pl.ds(t * TILE, TILE)], bufs.at[t], sems.at[t])
        copies[t].start()
    tail_cp = pltpu.make_async_copy(
        w2_hbm.at[pl.ds(NTILE * TILE, TAIL)], tail_v, sems.at[NBUF + 1])
    tail_cp.start()

    for c in range(CONTEXT):
        gcp[c].wait()
    w1_cp.wait()
    hid = jnp.maximum(_nt_dot(emb_v[...], w1_v[...]) + b1_ref[...], 0.0)

    m = jnp.full((1, 1), -jnp.inf, jnp.float32)
    s = jnp.zeros((1, 1), jnp.float32)
    for t in range(NTILE):
        copies[t].wait()
        x_t = _nt_dot(hid, bufs[t % NBUF]) + b2_ref[
            pl.ds(t * TILE, TILE)].reshape(1, TILE)
        out_ref[:, pl.ds(t * TILE, TILE)] = x_t
        nt = t + NBUF
        if nt < NTILE:
            copies[nt] = pltpu.make_async_copy(
                w2_hbm.at[pl.ds(nt * TILE, TILE)], bufs.at[nt % NBUF],
                sems.at[nt % NBUF])
            copies[nt].start()
        m_new = jnp.maximum(m, jnp.max(x_t))
        s = s * jnp.exp(m - m_new) + jnp.sum(
            jnp.exp(x_t - m_new), keepdims=True).reshape(1, 1)
        m = m_new

    tail_cp.wait()
    x_t = _nt_dot(hid, tail_v[...]) + b2_ref[
        pl.ds(NTILE * TILE, TAIL)].reshape(1, TAIL)
    out_ref[:, pl.ds(NTILE * TILE, TAIL)] = x_t
    m_new = jnp.maximum(m, jnp.max(x_t))
    s = s * jnp.exp(m - m_new) + jnp.sum(
        jnp.exp(x_t - m_new), keepdims=True).reshape(1, 1)
    m = m_new

    lse = m + jnp.log(s)
    out_ref[...] = out_ref[...] - lse


@jax.jit
def _tc_mlp(idx, b1, b2, emb, W1, W2):
    return pl.pallas_call(
        _mlp_kernel,
        in_specs=[
            pl.BlockSpec(memory_space=pltpu.MemorySpace.SMEM),
            pl.BlockSpec(memory_space=pltpu.MemorySpace.VMEM),
            pl.BlockSpec(memory_space=pltpu.MemorySpace.VMEM),
            pl.BlockSpec(memory_space=pltpu.MemorySpace.HBM),
            pl.BlockSpec(memory_space=pltpu.MemorySpace.HBM),
            pl.BlockSpec(memory_space=pltpu.MemorySpace.HBM),
        ],
        out_specs=pl.BlockSpec(memory_space=pltpu.MemorySpace.VMEM),
        out_shape=jax.ShapeDtypeStruct((1, VOCAB), jnp.float32),
        scratch_shapes=[
            pltpu.VMEM((1, KDIM), jnp.float32),
            pltpu.VMEM((HIDDEN, KDIM), jnp.float32),
            pltpu.VMEM((NBUF, TILE, EMBED_DIM), jnp.float32),
            pltpu.VMEM((TAIL, EMBED_DIM), jnp.float32),
            pltpu.SemaphoreType.DMA((NBUF + 2,)),
            pltpu.SemaphoreType.DMA,
        ],
    )(idx, b1, b2, emb, W1, W2)


def kernel(inputs, emb, W1, b1, W2, b2):
    return _tc_mlp(inputs.astype(jnp.int32), b1.reshape(1, HIDDEN),
                   b2, emb, W1, W2)
